# baseline (speedup 1.0000x reference)
import functools

import jax
import jax.numpy as jnp
from jax import lax
from jax.experimental import pallas as pl
from jax.experimental.pallas import tpu as pltpu
from jax.experimental.pallas import tpu_sc as plsc

D = 256
SCALE = 4
C = D // SCALE
V = C * C * C
ROW = 16
CHUNK = 2048
LANES = 16

NC = 2
NS = 16
NW = NC * NS


def _sc_body(n_chunks, xs, cells, out,
             xs_v, f0_v, f1_v, f2_v, idx_v, rows_v, out_v,
             sem):
    wid = lax.axis_index("s") * NC + lax.axis_index("c")
    base_w = wid * (n_chunks * CHUNK)
    iota = lax.iota(jnp.int32, LANES)
    col0 = jnp.zeros((LANES,), jnp.int32)
    col1 = col0 + 1
    col2 = col0 + 2

    def chunk_body(ch, _):
        base = base_w + ch * CHUNK
        pltpu.sync_copy(xs.at[pl.ds(base, CHUNK), :], xs_v)

        def idx_body(i, _):
            sl = pl.ds(i * LANES, LANES)
            p = i * LANES + iota
            s0 = plsc.load_gather(xs_v, [p, col0]) * jnp.float32(D)
            s1 = plsc.load_gather(xs_v, [p, col1]) * jnp.float32(D)
            s2 = plsc.load_gather(xs_v, [p, col2]) * jnp.float32(D)
            i0 = lax.bitwise_and(s0.astype(jnp.int32), -SCALE)
            i1 = lax.bitwise_and(s1.astype(jnp.int32), -SCALE)
            i2 = lax.bitwise_and(s2.astype(jnp.int32), -SCALE)
            f0_v[sl] = (s0 - i0.astype(jnp.float32)) * jnp.float32(1.0 / SCALE)
            f1_v[sl] = (s1 - i1.astype(jnp.float32)) * jnp.float32(1.0 / SCALE)
            f2_v[sl] = (s2 - i2.astype(jnp.float32)) * jnp.float32(1.0 / SCALE)
            c0 = lax.shift_right_arithmetic(i0, 2)
            c1 = lax.shift_right_arithmetic(i1, 2)
            c2 = lax.shift_right_arithmetic(i2, 2)
            idx_v[sl] = (c0 * C + c1) * C + c2
            return 0

        lax.fori_loop(0, CHUNK // LANES, idx_body, 0)

        copies = []
        for j in range(CHUNK // 128):
            isl = pl.ds(j * 128, 128)
            copies.append(
                pltpu.async_copy(cells.at[idx_v.at[isl]],
                                 rows_v.at[isl, :], sem))
        for cp in copies:
            cp.wait()


        def comp_body(i, _):
            sl = pl.ds(i * LANES, LANES)
            f0 = f0_v[sl]
            f1 = f1_v[sl]
            f2 = f2_v[sl]
            g0 = jnp.float32(1.0) - f0
            g1 = jnp.float32(1.0) - f1
            g2 = jnp.float32(1.0) - f2
            q00 = g1 * g2
            q10 = f1 * g2
            q01 = g1 * f2
            q11 = f1 * f2
            row = i * LANES + iota

            def corner(j):
                col = jnp.full((LANES,), j, jnp.int32)
                return plsc.load_gather(rows_v, [row, col])

            acc = corner(0) * (g0 * q00)
            acc = acc + corner(1) * (f0 * q00)
            acc = acc + corner(2) * (g0 * q10)
            acc = acc + corner(3) * (f0 * q10)
            acc = acc + corner(4) * (g0 * q01)
            acc = acc + corner(5) * (f0 * q01)
            acc = acc + corner(6) * (g0 * q11)
            acc = acc + corner(7) * (f0 * q11)
            out_v[sl] = acc
            return 0

        lax.fori_loop(0, CHUNK // LANES, comp_body, 0)
        pltpu.sync_copy(out_v, out.at[pl.ds(base, CHUNK)])
        return 0

    lax.fori_loop(0, n_chunks, chunk_body, 0)


def kernel(xs, data):
    n = xs.shape[0]
    n_chunks = n // (NW * CHUNK)

    d3 = data[:, :, :, 0]
    a = jnp.concatenate([d3[::SCALE], d3[D - 1:]], 0)
    b = jnp.concatenate([a[:, ::SCALE], a[:, D - 1:]], 1)
    c = jnp.concatenate([b[:, :, ::SCALE], b[:, :, D - 1:]], 2)
    corners = [c[dz:dz + C, dy:dy + C, dx:dx + C]
               for dz in (0, 1) for dy in (0, 1) for dx in (0, 1)]
    cells = jnp.stack(corners + corners[:8], axis=-1).reshape(V, ROW)

    mesh = plsc.VectorSubcoreMesh(core_axis_name="c", subcore_axis_name="s")
    run = functools.partial(
        pl.kernel,
        mesh=mesh,
        compiler_params=pltpu.CompilerParams(
            needs_layout_passes=False, use_tc_tiling_on_sc=False),
        out_type=jax.ShapeDtypeStruct((n,), jnp.float32),
        scratch_types=[
            pltpu.VMEM((CHUNK, 3), jnp.float32),
            pltpu.VMEM((CHUNK,), jnp.float32),
            pltpu.VMEM((CHUNK,), jnp.float32),
            pltpu.VMEM((CHUNK,), jnp.float32),
            pltpu.VMEM((CHUNK,), jnp.int32),
            pltpu.VMEM((CHUNK, ROW), jnp.float32),
            pltpu.VMEM((CHUNK,), jnp.float32),
            pltpu.SemaphoreType.DMA,
        ],
    )(functools.partial(_sc_body, n_chunks))
    out = run(xs, cells)
    return out.reshape(n, 1)


# speedup vs baseline: 1.1791x; 1.1791x over previous
import functools

import jax
import jax.numpy as jnp
from jax import lax
from jax.experimental import pallas as pl
from jax.experimental.pallas import tpu as pltpu
from jax.experimental.pallas import tpu_sc as plsc

D = 256
SCALE = 4
C = D // SCALE
V = C * C * C
ROW = 16
CHUNK = 2048
LANES = 16

NC = 2
NS = 16
NW = NC * NS


def _sc_body(n_chunks, xs, cells, out,
             xs_v, f0_v, f1_v, f2_v, idx_v, rows_v, out_v,
             sem):
    wid = lax.axis_index("s") * NC + lax.axis_index("c")
    base_w = wid * (n_chunks * CHUNK)
    iota = lax.iota(jnp.int32, LANES)

    def chunk_body(ch, _):
        base = base_w + ch * CHUNK
        pltpu.sync_copy(xs.at[pl.ds(base * 3, CHUNK * 3)], xs_v)

        def idx_body(i, _):
            sl = pl.ds(i * LANES, LANES)
            p = (i * LANES + iota) * 3
            s0 = plsc.load_gather(xs_v, [p]) * jnp.float32(D)
            s1 = plsc.load_gather(xs_v, [p + 1]) * jnp.float32(D)
            s2 = plsc.load_gather(xs_v, [p + 2]) * jnp.float32(D)
            i0 = lax.bitwise_and(s0.astype(jnp.int32), -SCALE)
            i1 = lax.bitwise_and(s1.astype(jnp.int32), -SCALE)
            i2 = lax.bitwise_and(s2.astype(jnp.int32), -SCALE)
            f0_v[sl] = (s0 - i0.astype(jnp.float32)) * jnp.float32(1.0 / SCALE)
            f1_v[sl] = (s1 - i1.astype(jnp.float32)) * jnp.float32(1.0 / SCALE)
            f2_v[sl] = (s2 - i2.astype(jnp.float32)) * jnp.float32(1.0 / SCALE)
            c0 = lax.shift_right_arithmetic(i0, 2)
            c1 = lax.shift_right_arithmetic(i1, 2)
            c2 = lax.shift_right_arithmetic(i2, 2)
            idx_v[sl] = (c0 * C + c1) * C + c2
            return 0

        lax.fori_loop(0, CHUNK // LANES, idx_body, 0)

        copies = []
        for j in range(CHUNK // 128):
            isl = pl.ds(j * 128, 128)
            copies.append(
                pltpu.async_copy(cells.at[idx_v.at[isl]],
                                 rows_v.at[isl, :], sem))
        for cp in copies:
            cp.wait()


        def comp_body(i, _):
            sl = pl.ds(i * LANES, LANES)
            f0 = f0_v[sl]
            f1 = f1_v[sl]
            f2 = f2_v[sl]
            g0 = jnp.float32(1.0) - f0
            g1 = jnp.float32(1.0) - f1
            g2 = jnp.float32(1.0) - f2
            q00 = g1 * g2
            q10 = f1 * g2
            q01 = g1 * f2
            q11 = f1 * f2
            row = i * LANES + iota

            def corner(j):
                col = jnp.full((LANES,), j, jnp.int32)
                return plsc.load_gather(rows_v, [row, col])

            acc = corner(0) * (g0 * q00)
            acc = acc + corner(1) * (f0 * q00)
            acc = acc + corner(2) * (g0 * q10)
            acc = acc + corner(3) * (f0 * q10)
            acc = acc + corner(4) * (g0 * q01)
            acc = acc + corner(5) * (f0 * q01)
            acc = acc + corner(6) * (g0 * q11)
            acc = acc + corner(7) * (f0 * q11)
            out_v[sl] = acc
            return 0

        lax.fori_loop(0, CHUNK // LANES, comp_body, 0)
        pltpu.sync_copy(out_v, out.at[pl.ds(base, CHUNK)])
        return 0

    lax.fori_loop(0, n_chunks, chunk_body, 0)


def kernel(xs, data):
    n = xs.shape[0]
    n_chunks = n // (NW * CHUNK)

    d3 = data[:, :, :, 0]
    a = jnp.concatenate([d3[::SCALE], d3[D - 1:]], 0)
    b = jnp.concatenate([a[:, ::SCALE], a[:, D - 1:]], 1)
    c = jnp.concatenate([b[:, :, ::SCALE], b[:, :, D - 1:]], 2)
    corners = [c[dz:dz + C, dy:dy + C, dx:dx + C]
               for dz in (0, 1) for dy in (0, 1) for dx in (0, 1)]
    cells = jnp.stack(corners + corners[:8], axis=-1).reshape(V, ROW)

    mesh = plsc.VectorSubcoreMesh(core_axis_name="c", subcore_axis_name="s")
    run = functools.partial(
        pl.kernel,
        mesh=mesh,
        compiler_params=pltpu.CompilerParams(
            needs_layout_passes=False, use_tc_tiling_on_sc=False),
        out_type=jax.ShapeDtypeStruct((n,), jnp.float32),
        scratch_types=[
            pltpu.VMEM((CHUNK * 3,), jnp.float32),
            pltpu.VMEM((CHUNK,), jnp.float32),
            pltpu.VMEM((CHUNK,), jnp.float32),
            pltpu.VMEM((CHUNK,), jnp.float32),
            pltpu.VMEM((CHUNK,), jnp.int32),
            pltpu.VMEM((CHUNK, ROW), jnp.float32),
            pltpu.VMEM((CHUNK,), jnp.float32),
            pltpu.SemaphoreType.DMA,
        ],
    )(functools.partial(_sc_body, n_chunks))
    out = run(xs.reshape(n * 3), cells)
    return out.reshape(n, 1)


# speedup vs baseline: 4.0774x; 3.4580x over previous
import functools

import jax
import jax.numpy as jnp
from jax import lax
from jax.experimental import pallas as pl
from jax.experimental.pallas import tpu as pltpu
from jax.experimental.pallas import tpu_sc as plsc

D = 256
SCALE = 4
C = D // SCALE
V = C * C * C
ROW = 16
CHUNK = 2048
LANES = 16

NC = 2
NS = 16
NW = NC * NS


def _sc_body(n_chunks, xs0, xs1, xs2, cells, out,
             x0_v, x1_v, x2_v, f0_v, f1_v, f2_v, idx_v, rows_v, out_v,
             sem):
    wid = lax.axis_index("s") * NC + lax.axis_index("c")
    base_w = wid * (n_chunks * CHUNK)
    iota = lax.iota(jnp.int32, LANES)

    def chunk_body(ch, _):
        base = base_w + ch * CHUNK
        pltpu.sync_copy(xs0.at[pl.ds(base, CHUNK)], x0_v)
        pltpu.sync_copy(xs1.at[pl.ds(base, CHUNK)], x1_v)
        pltpu.sync_copy(xs2.at[pl.ds(base, CHUNK)], x2_v)

        def idx_body(i, _):
            sl = pl.ds(i * LANES, LANES)
            s0 = x0_v[sl] * jnp.float32(D)
            s1 = x1_v[sl] * jnp.float32(D)
            s2 = x2_v[sl] * jnp.float32(D)
            i0 = lax.bitwise_and(s0.astype(jnp.int32), -SCALE)
            i1 = lax.bitwise_and(s1.astype(jnp.int32), -SCALE)
            i2 = lax.bitwise_and(s2.astype(jnp.int32), -SCALE)
            f0_v[sl] = (s0 - i0.astype(jnp.float32)) * jnp.float32(1.0 / SCALE)
            f1_v[sl] = (s1 - i1.astype(jnp.float32)) * jnp.float32(1.0 / SCALE)
            f2_v[sl] = (s2 - i2.astype(jnp.float32)) * jnp.float32(1.0 / SCALE)
            c0 = lax.shift_right_arithmetic(i0, 2)
            c1 = lax.shift_right_arithmetic(i1, 2)
            c2 = lax.shift_right_arithmetic(i2, 2)
            idx_v[sl] = (c0 * C + c1) * C + c2
            return 0

        lax.fori_loop(0, CHUNK // LANES, idx_body, 0)

        copies = []
        for j in range(CHUNK // 128):
            isl = pl.ds(j * 128, 128)
            copies.append(
                pltpu.async_copy(cells.at[idx_v.at[isl]],
                                 rows_v.at[isl, :], sem))
        for cp in copies:
            cp.wait()


        def comp_body(i, _):
            sl = pl.ds(i * LANES, LANES)
            f0 = f0_v[sl]
            f1 = f1_v[sl]
            f2 = f2_v[sl]
            g0 = jnp.float32(1.0) - f0
            g1 = jnp.float32(1.0) - f1
            g2 = jnp.float32(1.0) - f2
            q00 = g1 * g2
            q10 = f1 * g2
            q01 = g1 * f2
            q11 = f1 * f2
            row = i * LANES + iota

            def corner(j):
                col = jnp.full((LANES,), j, jnp.int32)
                return plsc.load_gather(rows_v, [row, col])

            acc = corner(0) * (g0 * q00)
            acc = acc + corner(1) * (f0 * q00)
            acc = acc + corner(2) * (g0 * q10)
            acc = acc + corner(3) * (f0 * q10)
            acc = acc + corner(4) * (g0 * q01)
            acc = acc + corner(5) * (f0 * q01)
            acc = acc + corner(6) * (g0 * q11)
            acc = acc + corner(7) * (f0 * q11)
            out_v[sl] = acc
            return 0

        lax.fori_loop(0, CHUNK // LANES, comp_body, 0)
        pltpu.sync_copy(out_v, out.at[pl.ds(base, CHUNK)])
        return 0

    lax.fori_loop(0, n_chunks, chunk_body, 0)


def kernel(xs, data):
    n = xs.shape[0]
    n_chunks = n // (NW * CHUNK)

    d3 = data[:, :, :, 0]
    a = jnp.concatenate([d3[::SCALE], d3[D - 1:]], 0)
    b = jnp.concatenate([a[:, ::SCALE], a[:, D - 1:]], 1)
    c = jnp.concatenate([b[:, :, ::SCALE], b[:, :, D - 1:]], 2)
    corners = [c[dz:dz + C, dy:dy + C, dx:dx + C]
               for dz in (0, 1) for dy in (0, 1) for dx in (0, 1)]
    cells = jnp.stack(corners + corners[:8], axis=-1).reshape(V, ROW)

    mesh = plsc.VectorSubcoreMesh(core_axis_name="c", subcore_axis_name="s")
    run = functools.partial(
        pl.kernel,
        mesh=mesh,
        compiler_params=pltpu.CompilerParams(
            needs_layout_passes=False, use_tc_tiling_on_sc=False),
        out_type=jax.ShapeDtypeStruct((n,), jnp.float32),
        scratch_types=[
            pltpu.VMEM((CHUNK,), jnp.float32),
            pltpu.VMEM((CHUNK,), jnp.float32),
            pltpu.VMEM((CHUNK,), jnp.float32),
            pltpu.VMEM((CHUNK,), jnp.float32),
            pltpu.VMEM((CHUNK,), jnp.float32),
            pltpu.VMEM((CHUNK,), jnp.float32),
            pltpu.VMEM((CHUNK,), jnp.int32),
            pltpu.VMEM((CHUNK, ROW), jnp.float32),
            pltpu.VMEM((CHUNK,), jnp.float32),
            pltpu.SemaphoreType.DMA,
        ],
    )(functools.partial(_sc_body, n_chunks))
    out = run(xs[:, 0], xs[:, 1], xs[:, 2], cells)
    return out.reshape(n, 1)


# speedup vs baseline: 5.4504x; 1.3367x over previous
import functools

import jax
import jax.numpy as jnp
from jax import lax
from jax.experimental import pallas as pl
from jax.experimental.pallas import tpu as pltpu
from jax.experimental.pallas import tpu_sc as plsc

D = 256
SCALE = 4
C = D // SCALE
V = C * C * C
ROW = 16
CHUNK = 2048
LANES = 16

NC = 2
NS = 16
NW = NC * NS

CP = 65 * 65
CPP = 4232
CZ_PER_TILE = C // NS

CORNER_OFF = [dz * CPP + dy * 65 + dx
              for dz in (0, 1) for dy in (0, 1) for dx in (0, 1)]


def _sc_body(n_chunks, xs0, xs1, xs2, coarse, out,
             x0_v, x1_v, x2_v, f0_v, f1_v, f2_v, idx_v, rows_v, out_v,
             cslab_v, slab_v, cells, sem):
    cid = lax.axis_index("c")
    sid = lax.axis_index("s")
    wid = sid * NC + cid
    iota = lax.iota(jnp.int32, LANES)

    def build_cz(t, _):
        cz = sid * CZ_PER_TILE + t
        pltpu.sync_copy(coarse.at[pl.ds(cz * CPP, 2 * CPP)], cslab_v)

        def build_cy(cy, _):
            for g in range(C // LANES):
                cx = g * LANES + iota
                pos = cy * 65 + cx
                rowi = cy * C + cx
                for j in range(8):
                    vals = plsc.load_gather(cslab_v, [pos + CORNER_OFF[j]])
                    plsc.store_scatter(slab_v, [rowi, jnp.full((LANES,), j,
                                                               jnp.int32)],
                                       vals)
            return 0

        lax.fori_loop(0, C, build_cy, 0)
        pltpu.sync_copy(slab_v, cells.at[pl.ds(cz * (C * C), C * C), :])
        return 0

    lax.fori_loop(0, CZ_PER_TILE, build_cz, 0)
    plsc.subcore_barrier()

    base_w = wid * (n_chunks * CHUNK)

    def chunk_body(ch, _):
        base = base_w + ch * CHUNK
        pltpu.sync_copy(xs0.at[pl.ds(base, CHUNK)], x0_v)
        pltpu.sync_copy(xs1.at[pl.ds(base, CHUNK)], x1_v)
        pltpu.sync_copy(xs2.at[pl.ds(base, CHUNK)], x2_v)

        def idx_body(i, _):
            sl = pl.ds(i * LANES, LANES)
            s0 = x0_v[sl] * jnp.float32(D)
            s1 = x1_v[sl] * jnp.float32(D)
            s2 = x2_v[sl] * jnp.float32(D)
            i0 = lax.bitwise_and(s0.astype(jnp.int32), -SCALE)
            i1 = lax.bitwise_and(s1.astype(jnp.int32), -SCALE)
            i2 = lax.bitwise_and(s2.astype(jnp.int32), -SCALE)
            f0_v[sl] = (s0 - i0.astype(jnp.float32)) * jnp.float32(1.0 / SCALE)
            f1_v[sl] = (s1 - i1.astype(jnp.float32)) * jnp.float32(1.0 / SCALE)
            f2_v[sl] = (s2 - i2.astype(jnp.float32)) * jnp.float32(1.0 / SCALE)
            c0 = lax.shift_right_arithmetic(i0, 2)
            c1 = lax.shift_right_arithmetic(i1, 2)
            c2 = lax.shift_right_arithmetic(i2, 2)
            idx_v[sl] = (c0 * C + c1) * C + c2
            return 0

        lax.fori_loop(0, CHUNK // LANES, idx_body, 0)

        copies = []
        for j in range(CHUNK // 128):
            isl = pl.ds(j * 128, 128)
            copies.append(
                pltpu.async_copy(cells.at[idx_v.at[isl]],
                                 rows_v.at[isl, :], sem))
        for cp in copies:
            cp.wait()

        def comp_body(i, _):
            sl = pl.ds(i * LANES, LANES)
            f0 = f0_v[sl]
            f1 = f1_v[sl]
            f2 = f2_v[sl]
            g0 = jnp.float32(1.0) - f0
            g1 = jnp.float32(1.0) - f1
            g2 = jnp.float32(1.0) - f2
            q00 = g1 * g2
            q10 = f1 * g2
            q01 = g1 * f2
            q11 = f1 * f2
            row = i * LANES + iota

            def corner(j):
                col = jnp.full((LANES,), j, jnp.int32)
                return plsc.load_gather(rows_v, [row, col])

            acc = corner(0) * (g0 * q00)
            acc = acc + corner(1) * (f0 * q00)
            acc = acc + corner(2) * (g0 * q10)
            acc = acc + corner(3) * (f0 * q10)
            acc = acc + corner(4) * (g0 * q01)
            acc = acc + corner(5) * (f0 * q01)
            acc = acc + corner(6) * (g0 * q11)
            acc = acc + corner(7) * (f0 * q11)
            out_v[sl] = acc
            return 0

        lax.fori_loop(0, CHUNK // LANES, comp_body, 0)
        pltpu.sync_copy(out_v, out.at[pl.ds(base, CHUNK)])
        return 0

    lax.fori_loop(0, n_chunks, chunk_body, 0)


def kernel(xs, data):
    n = xs.shape[0]
    n_chunks = n // (NW * CHUNK)

    d3 = data[:, :, :, 0]
    a = jnp.concatenate([d3[::SCALE], d3[D - 1:]], 0)
    b = jnp.concatenate([a[:, ::SCALE], a[:, D - 1:]], 1)
    c = jnp.concatenate([b[:, :, ::SCALE], b[:, :, D - 1:]], 2)
    cflat = jnp.pad(c.reshape(65, CP), ((0, 1), (0, CPP - CP))).reshape(-1)

    mesh = plsc.VectorSubcoreMesh(core_axis_name="c", subcore_axis_name="s")
    run = functools.partial(
        pl.kernel,
        mesh=mesh,
        compiler_params=pltpu.CompilerParams(
            needs_layout_passes=False, use_tc_tiling_on_sc=False),
        out_type=jax.ShapeDtypeStruct((n,), jnp.float32),
        scratch_types=[
            pltpu.VMEM((CHUNK,), jnp.float32),
            pltpu.VMEM((CHUNK,), jnp.float32),
            pltpu.VMEM((CHUNK,), jnp.float32),
            pltpu.VMEM((CHUNK,), jnp.float32),
            pltpu.VMEM((CHUNK,), jnp.float32),
            pltpu.VMEM((CHUNK,), jnp.float32),
            pltpu.VMEM((CHUNK,), jnp.int32),
            pltpu.VMEM((CHUNK, ROW), jnp.float32),
            pltpu.VMEM((CHUNK,), jnp.float32),
            pltpu.VMEM((2 * CPP,), jnp.float32),
            pltpu.VMEM((C * C, ROW), jnp.float32),
            pltpu.HBM((V, ROW), jnp.float32),
            pltpu.SemaphoreType.DMA,
        ],
    )(functools.partial(_sc_body, n_chunks))
    out = run(xs[:, 0], xs[:, 1], xs[:, 2], cflat)
    return out.reshape(n, 1)


# speedup vs baseline: 6.5328x; 1.1986x over previous
import functools

import jax
import jax.numpy as jnp
from jax import lax
from jax.experimental import pallas as pl
from jax.experimental.pallas import tpu as pltpu
from jax.experimental.pallas import tpu_sc as plsc

D = 256
SCALE = 4
C = D // SCALE
V = C * C * C
ROW = 8
CHUNK = 2048
LANES = 16

NC = 2
NS = 16
NW = NC * NS

CP = 65 * 65
CPP = 4232
CZ_PER_TILE = C // NS

CORNER_OFF = [dz * CPP + dy * 65 + dx
              for dz in (0, 1) for dy in (0, 1) for dx in (0, 1)]


def _sc_body(n_chunks, xs0, xs1, xs2, coarse, out,
             x0_v, x1_v, x2_v, f0_v, f1_v, f2_v, idx_v, rows_v, out_v,
             cslab_v, slab_v, cells, sem_a, sem_b):
    cid = lax.axis_index("c")
    sid = lax.axis_index("s")
    wid = sid * NC + cid
    iota = lax.iota(jnp.int32, LANES)

    def build_cz(t, _):
        cz = sid * CZ_PER_TILE + t
        pltpu.sync_copy(coarse.at[pl.ds(cz * CPP, 2 * CPP)], cslab_v)

        def build_cy(cy, _):
            for g in range(C // LANES):
                cx = g * LANES + iota
                pos = cy * 65 + cx
                rowi = cy * C + cx
                for j in range(8):
                    vals = plsc.load_gather(cslab_v, [pos + CORNER_OFF[j]])
                    plsc.store_scatter(slab_v, [rowi, jnp.full((LANES,), j,
                                                               jnp.int32)],
                                       vals)
            return 0

        lax.fori_loop(0, C, build_cy, 0)
        pltpu.sync_copy(slab_v, cells.at[pl.ds(cz * (C * C), C * C), :])
        return 0

    lax.fori_loop(0, CZ_PER_TILE, build_cz, 0)
    plsc.subcore_barrier()

    base_w = wid * (n_chunks * CHUNK)
    sems = (sem_a, sem_b)

    def stage_a(ch):
        b = (ch % 2) * CHUNK
        base = base_w + ch * CHUNK
        pltpu.sync_copy(xs0.at[pl.ds(base, CHUNK)], x0_v.at[pl.ds(b, CHUNK)])
        pltpu.sync_copy(xs1.at[pl.ds(base, CHUNK)], x1_v.at[pl.ds(b, CHUNK)])
        pltpu.sync_copy(xs2.at[pl.ds(base, CHUNK)], x2_v.at[pl.ds(b, CHUNK)])

        def idx_body(i, _):
            sl = pl.ds(b + i * LANES, LANES)
            s0 = x0_v[sl] * jnp.float32(D)
            s1 = x1_v[sl] * jnp.float32(D)
            s2 = x2_v[sl] * jnp.float32(D)
            i0 = lax.bitwise_and(s0.astype(jnp.int32), -SCALE)
            i1 = lax.bitwise_and(s1.astype(jnp.int32), -SCALE)
            i2 = lax.bitwise_and(s2.astype(jnp.int32), -SCALE)
            f0_v[sl] = (s0 - i0.astype(jnp.float32)) * jnp.float32(1.0 / SCALE)
            f1_v[sl] = (s1 - i1.astype(jnp.float32)) * jnp.float32(1.0 / SCALE)
            f2_v[sl] = (s2 - i2.astype(jnp.float32)) * jnp.float32(1.0 / SCALE)
            c0 = lax.shift_right_arithmetic(i0, 2)
            c1 = lax.shift_right_arithmetic(i1, 2)
            c2 = lax.shift_right_arithmetic(i2, 2)
            idx_v[sl] = (c0 * C + c1) * C + c2
            return 0

        lax.fori_loop(0, CHUNK // LANES, idx_body, 0)

        copies = []
        for j in range(CHUNK // 128):
            isl = pl.ds(b + j * 128, 128)
            copies.append(
                pltpu.async_copy(cells.at[idx_v.at[isl]],
                                 rows_v.at[isl, :], sems[ch % 2]))
        return copies

    def stage_b(ch, copies):
        b = (ch % 2) * CHUNK
        base = base_w + ch * CHUNK
        for cp in copies:
            cp.wait()

        def comp_body(i, _):
            sl = pl.ds(b + i * LANES, LANES)
            f0 = f0_v[sl]
            f1 = f1_v[sl]
            f2 = f2_v[sl]
            g0 = jnp.float32(1.0) - f0
            g1 = jnp.float32(1.0) - f1
            g2 = jnp.float32(1.0) - f2
            q00 = g1 * g2
            q10 = f1 * g2
            q01 = g1 * f2
            q11 = f1 * f2
            row = b + i * LANES + iota

            def corner(j):
                col = jnp.full((LANES,), j, jnp.int32)
                return plsc.load_gather(rows_v, [row, col])

            acc = corner(0) * (g0 * q00)
            acc = acc + corner(1) * (f0 * q00)
            acc = acc + corner(2) * (g0 * q10)
            acc = acc + corner(3) * (f0 * q10)
            acc = acc + corner(4) * (g0 * q01)
            acc = acc + corner(5) * (f0 * q01)
            acc = acc + corner(6) * (g0 * q11)
            acc = acc + corner(7) * (f0 * q11)
            out_v[pl.ds(i * LANES, LANES)] = acc
            return 0

        lax.fori_loop(0, CHUNK // LANES, comp_body, 0)
        pltpu.sync_copy(out_v, out.at[pl.ds(base, CHUNK)])

    inflight = stage_a(0)
    for ch in range(n_chunks):
        nxt = stage_a(ch + 1) if ch + 1 < n_chunks else None
        stage_b(ch, inflight)
        inflight = nxt


def kernel(xs, data):
    n = xs.shape[0]
    n_chunks = n // (NW * CHUNK)

    d3 = data[:, :, :, 0]
    a = jnp.concatenate([d3[::SCALE], d3[D - 1:]], 0)
    b = jnp.concatenate([a[:, ::SCALE], a[:, D - 1:]], 1)
    c = jnp.concatenate([b[:, :, ::SCALE], b[:, :, D - 1:]], 2)
    cflat = jnp.pad(c.reshape(65, CP), ((0, 1), (0, CPP - CP))).reshape(-1)

    mesh = plsc.VectorSubcoreMesh(core_axis_name="c", subcore_axis_name="s")
    run = functools.partial(
        pl.kernel,
        mesh=mesh,
        compiler_params=pltpu.CompilerParams(
            needs_layout_passes=False, use_tc_tiling_on_sc=False),
        out_type=jax.ShapeDtypeStruct((n,), jnp.float32),
        scratch_types=[
            pltpu.VMEM((2 * CHUNK,), jnp.float32),
            pltpu.VMEM((2 * CHUNK,), jnp.float32),
            pltpu.VMEM((2 * CHUNK,), jnp.float32),
            pltpu.VMEM((2 * CHUNK,), jnp.float32),
            pltpu.VMEM((2 * CHUNK,), jnp.float32),
            pltpu.VMEM((2 * CHUNK,), jnp.float32),
            pltpu.VMEM((2 * CHUNK,), jnp.int32),
            pltpu.VMEM((2 * CHUNK, ROW), jnp.float32),
            pltpu.VMEM((CHUNK,), jnp.float32),
            pltpu.VMEM((2 * CPP,), jnp.float32),
            pltpu.VMEM((C * C, ROW), jnp.float32),
            pltpu.HBM((V, ROW), jnp.float32),
            pltpu.SemaphoreType.DMA,
            pltpu.SemaphoreType.DMA,
        ],
    )(functools.partial(_sc_body, n_chunks))
    out = run(xs[:, 0], xs[:, 1], xs[:, 2], cflat)
    return out.reshape(n, 1)


# speedup vs baseline: 6.8247x; 1.0447x over previous
import functools

import jax
import jax.numpy as jnp
from jax import lax
from jax.experimental import pallas as pl
from jax.experimental.pallas import tpu as pltpu
from jax.experimental.pallas import tpu_sc as plsc

D = 256
SCALE = 4
C = D // SCALE
V = C * C * C
ROW = 8
CHUNK = 2048
LANES = 16

NC = 2
NS = 16
NW = NC * NS

CP = 65 * 65
CPP = 4232
CZ_PER_TILE = C // NS

CORNER_OFF = [dz * CPP + dy * 65 + dx
              for dz in (0, 1) for dy in (0, 1) for dx in (0, 1)]


def _sc_body(n_chunks, xs0, xs1, xs2, coarse, out,
             x0_v, x1_v, x2_v, f0_v, f1_v, f2_v, idx_v, rows_v, out_v,
             cslab_v, slab_v, cells, sem_a, sem_b):
    cid = lax.axis_index("c")
    sid = lax.axis_index("s")
    wid = sid * NC + cid
    iota = lax.iota(jnp.int32, LANES)

    def build_cz(t, _):
        cz = sid * CZ_PER_TILE + t
        pltpu.sync_copy(coarse.at[pl.ds(cz * CPP, 2 * CPP)], cslab_v)

        def build_cy(cy, _):
            for g in range(C // LANES):
                cx = g * LANES + iota
                pos = cy * 65 + cx
                rowi = cy * C + cx
                for j in range(8):
                    vals = plsc.load_gather(cslab_v, [pos + CORNER_OFF[j]])
                    plsc.store_scatter(slab_v, [rowi, jnp.full((LANES,), j,
                                                               jnp.int32)],
                                       vals)
            return 0

        lax.fori_loop(0, C, build_cy, 0)
        pltpu.sync_copy(slab_v, cells.at[pl.ds(cz * (C * C), C * C), :])
        return 0

    lax.fori_loop(0, CZ_PER_TILE, build_cz, 0)
    plsc.subcore_barrier()

    base_w = wid * (n_chunks * CHUNK)
    sems = (sem_a, sem_b)

    def stage_a(ch):
        b = (ch % 2) * CHUNK
        base = base_w + ch * CHUNK
        pltpu.sync_copy(xs0.at[pl.ds(base, CHUNK)], x0_v.at[pl.ds(b, CHUNK)])
        pltpu.sync_copy(xs1.at[pl.ds(base, CHUNK)], x1_v.at[pl.ds(b, CHUNK)])
        pltpu.sync_copy(xs2.at[pl.ds(base, CHUNK)], x2_v.at[pl.ds(b, CHUNK)])

        def idx_body(i, _):
            sl = pl.ds(b + i * LANES, LANES)
            s0 = x0_v[sl] * jnp.float32(D)
            s1 = x1_v[sl] * jnp.float32(D)
            s2 = x2_v[sl] * jnp.float32(D)
            i0 = lax.bitwise_and(s0.astype(jnp.int32), -SCALE)
            i1 = lax.bitwise_and(s1.astype(jnp.int32), -SCALE)
            i2 = lax.bitwise_and(s2.astype(jnp.int32), -SCALE)
            f0_v[sl] = (s0 - i0.astype(jnp.float32)) * jnp.float32(1.0 / SCALE)
            f1_v[sl] = (s1 - i1.astype(jnp.float32)) * jnp.float32(1.0 / SCALE)
            f2_v[sl] = (s2 - i2.astype(jnp.float32)) * jnp.float32(1.0 / SCALE)
            c0 = lax.shift_right_arithmetic(i0, 2)
            c1 = lax.shift_right_arithmetic(i1, 2)
            c2 = lax.shift_right_arithmetic(i2, 2)
            idx_v[sl] = (c0 * C + c1) * C + c2
            return 0

        lax.fori_loop(0, CHUNK // LANES, idx_body, 0)

        copies = []
        for j in range(CHUNK // 128):
            isl = pl.ds(b + j * 128, 128)
            copies.append(
                pltpu.async_copy(cells.at[idx_v.at[isl]],
                                 rows_v.at[isl, :], sems[ch % 2]))
        return copies

    def stage_b(ch, copies):
        b = (ch % 2) * CHUNK
        base = base_w + ch * CHUNK
        for cp in copies:
            cp.wait()

        def comp_body(i, _):
            sl = pl.ds(b + i * LANES, LANES)
            f0 = f0_v[sl]
            f1 = f1_v[sl]
            f2 = f2_v[sl]
            g0 = jnp.float32(1.0) - f0
            g1 = jnp.float32(1.0) - f1
            g2 = jnp.float32(1.0) - f2
            q00 = g1 * g2
            q10 = f1 * g2
            q01 = g1 * f2
            q11 = f1 * f2
            row = b + i * LANES + iota

            def corner(j):
                col = jnp.full((LANES,), j, jnp.int32)
                return plsc.load_gather(rows_v, [row, col])

            acc = corner(0) * (g0 * q00)
            acc = acc + corner(1) * (f0 * q00)
            acc = acc + corner(2) * (g0 * q10)
            acc = acc + corner(3) * (f0 * q10)
            acc = acc + corner(4) * (g0 * q01)
            acc = acc + corner(5) * (f0 * q01)
            acc = acc + corner(6) * (g0 * q11)
            acc = acc + corner(7) * (f0 * q11)
            out_v[pl.ds(i * LANES, LANES)] = acc
            return 0

        lax.fori_loop(0, CHUNK // LANES, comp_body, 0)
        pltpu.sync_copy(out_v, out.at[pl.ds(base, CHUNK)])

    inflight = stage_a(0)
    for ch in range(n_chunks):
        nxt = stage_a(ch + 1) if ch + 1 < n_chunks else None
        stage_b(ch, inflight)
        inflight = nxt


def kernel(xs, data):
    n = xs.shape[0]
    n_chunks = n // (NW * CHUNK)

    a = jnp.concatenate([data[::SCALE], data[D - 1:]], 0)[:, :, :, 0]
    b = jnp.concatenate([a[:, ::SCALE], a[:, D - 1:]], 1)
    c = jnp.concatenate([b[:, :, ::SCALE], b[:, :, D - 1:]], 2)
    cflat = jnp.pad(c.reshape(65, CP), ((0, 1), (0, CPP - CP))).reshape(-1)
    xst = xs.T

    mesh = plsc.VectorSubcoreMesh(core_axis_name="c", subcore_axis_name="s")
    run = functools.partial(
        pl.kernel,
        mesh=mesh,
        compiler_params=pltpu.CompilerParams(
            needs_layout_passes=False, use_tc_tiling_on_sc=False),
        out_type=jax.ShapeDtypeStruct((n,), jnp.float32),
        scratch_types=[
            pltpu.VMEM((2 * CHUNK,), jnp.float32),
            pltpu.VMEM((2 * CHUNK,), jnp.float32),
            pltpu.VMEM((2 * CHUNK,), jnp.float32),
            pltpu.VMEM((2 * CHUNK,), jnp.float32),
            pltpu.VMEM((2 * CHUNK,), jnp.float32),
            pltpu.VMEM((2 * CHUNK,), jnp.float32),
            pltpu.VMEM((2 * CHUNK,), jnp.int32),
            pltpu.VMEM((2 * CHUNK, ROW), jnp.float32),
            pltpu.VMEM((CHUNK,), jnp.float32),
            pltpu.VMEM((2 * CPP,), jnp.float32),
            pltpu.VMEM((C * C, ROW), jnp.float32),
            pltpu.HBM((V, ROW), jnp.float32),
            pltpu.SemaphoreType.DMA,
            pltpu.SemaphoreType.DMA,
        ],
    )(functools.partial(_sc_body, n_chunks))
    out = run(xst[0], xst[1], xst[2], cflat)
    return out.reshape(n, 1)


# speedup vs baseline: 7.5467x; 1.1058x over previous
import functools

import jax
import jax.numpy as jnp
from jax import lax
from jax.experimental import pallas as pl
from jax.experimental.pallas import tpu as pltpu
from jax.experimental.pallas import tpu_sc as plsc

D = 256
SCALE = 4
C = D // SCALE
V = C * C * C
ROW = 8
CHUNK = 2048
LANES = 16

NC = 2
NS = 16
NW = NC * NS

CP = 65 * 65
CPP = 4232
CZ_PER_TILE = C // NW

CORNER_OFF = [dz * CPP + dy * 65 + dx
              for dz in (0, 1) for dy in (0, 1) for dx in (0, 1)]


def _sc_build(coarse, cells, cslab_v, slab_v):
    wid = lax.axis_index("s") * NC + lax.axis_index("c")
    iota = lax.iota(jnp.int32, LANES)

    def build_cz(t, _):
        cz = wid * CZ_PER_TILE + t
        pltpu.sync_copy(coarse.at[pl.ds(cz * CPP, 2 * CPP)], cslab_v)

        def build_cy(cy, _):
            for g in range(C // LANES):
                cx = g * LANES + iota
                pos = cy * 65 + cx
                rowi = cy * C + cx
                for j in range(8):
                    vals = plsc.load_gather(cslab_v, [pos + CORNER_OFF[j]])
                    plsc.store_scatter(slab_v, [rowi, jnp.full((LANES,), j,
                                                               jnp.int32)],
                                       vals)
            return 0

        lax.fori_loop(0, C, build_cy, 0)
        pltpu.sync_copy(slab_v, cells.at[pl.ds(cz * (C * C), C * C), :])
        return 0

    lax.fori_loop(0, CZ_PER_TILE, build_cz, 0)


def _sc_body(n_chunks, xs0, xs1, xs2, cells, out,
             x0_v, x1_v, x2_v, f0_v, f1_v, f2_v, idx_v, rows_v, out_v,
             sem_a, sem_b):
    wid = lax.axis_index("s") * NC + lax.axis_index("c")
    iota = lax.iota(jnp.int32, LANES)

    base_w = wid * (n_chunks * CHUNK)
    sems = (sem_a, sem_b)

    def stage_a(ch):
        b = (ch % 2) * CHUNK
        base = base_w + ch * CHUNK
        pltpu.sync_copy(xs0.at[pl.ds(base, CHUNK)], x0_v.at[pl.ds(b, CHUNK)])
        pltpu.sync_copy(xs1.at[pl.ds(base, CHUNK)], x1_v.at[pl.ds(b, CHUNK)])
        pltpu.sync_copy(xs2.at[pl.ds(base, CHUNK)], x2_v.at[pl.ds(b, CHUNK)])

        def idx_body(i, _):
            sl = pl.ds(b + i * LANES, LANES)
            s0 = x0_v[sl] * jnp.float32(D)
            s1 = x1_v[sl] * jnp.float32(D)
            s2 = x2_v[sl] * jnp.float32(D)
            i0 = lax.bitwise_and(s0.astype(jnp.int32), -SCALE)
            i1 = lax.bitwise_and(s1.astype(jnp.int32), -SCALE)
            i2 = lax.bitwise_and(s2.astype(jnp.int32), -SCALE)
            f0_v[sl] = (s0 - i0.astype(jnp.float32)) * jnp.float32(1.0 / SCALE)
            f1_v[sl] = (s1 - i1.astype(jnp.float32)) * jnp.float32(1.0 / SCALE)
            f2_v[sl] = (s2 - i2.astype(jnp.float32)) * jnp.float32(1.0 / SCALE)
            c0 = lax.shift_right_arithmetic(i0, 2)
            c1 = lax.shift_right_arithmetic(i1, 2)
            c2 = lax.shift_right_arithmetic(i2, 2)
            idx_v[sl] = (c0 * C + c1) * C + c2
            return 0

        lax.fori_loop(0, CHUNK // LANES, idx_body, 0)

        copies = []
        for j in range(CHUNK // 128):
            isl = pl.ds(b + j * 128, 128)
            copies.append(
                pltpu.async_copy(cells.at[idx_v.at[isl]],
                                 rows_v.at[isl, :], sems[ch % 2]))
        return copies

    def stage_b(ch, copies):
        b = (ch % 2) * CHUNK
        base = base_w + ch * CHUNK
        for cp in copies:
            cp.wait()

        def comp_body(i, _):
            sl = pl.ds(b + i * LANES, LANES)
            f0 = f0_v[sl]
            f1 = f1_v[sl]
            f2 = f2_v[sl]
            g0 = jnp.float32(1.0) - f0
            g1 = jnp.float32(1.0) - f1
            g2 = jnp.float32(1.0) - f2
            q00 = g1 * g2
            q10 = f1 * g2
            q01 = g1 * f2
            q11 = f1 * f2
            row = b + i * LANES + iota

            def corner(j):
                col = jnp.full((LANES,), j, jnp.int32)
                return plsc.load_gather(rows_v, [row, col])

            acc = corner(0) * (g0 * q00)
            acc = acc + corner(1) * (f0 * q00)
            acc = acc + corner(2) * (g0 * q10)
            acc = acc + corner(3) * (f0 * q10)
            acc = acc + corner(4) * (g0 * q01)
            acc = acc + corner(5) * (f0 * q01)
            acc = acc + corner(6) * (g0 * q11)
            acc = acc + corner(7) * (f0 * q11)
            out_v[pl.ds(i * LANES, LANES)] = acc
            return 0

        lax.fori_loop(0, CHUNK // LANES, comp_body, 0)
        pltpu.sync_copy(out_v, out.at[pl.ds(base, CHUNK)])

    inflight = stage_a(0)
    for ch in range(n_chunks):
        nxt = stage_a(ch + 1) if ch + 1 < n_chunks else None
        stage_b(ch, inflight)
        inflight = nxt


def kernel(xs, data):
    n = xs.shape[0]
    n_chunks = n // (NW * CHUNK)

    a = jnp.concatenate([data[::SCALE], data[D - 1:]], 0)[:, :, :, 0]
    b = jnp.concatenate([a[:, ::SCALE], a[:, D - 1:]], 1)
    c = jnp.concatenate([b[:, :, ::SCALE], b[:, :, D - 1:]], 2)
    cflat = jnp.pad(c.reshape(65, CP), ((0, 1), (0, CPP - CP))).reshape(-1)
    xst = xs.T

    mesh = plsc.VectorSubcoreMesh(core_axis_name="c", subcore_axis_name="s")
    cp = pltpu.CompilerParams(
        needs_layout_passes=False, use_tc_tiling_on_sc=False)
    cells = pl.kernel(
        _sc_build,
        mesh=mesh,
        compiler_params=cp,
        out_type=jax.ShapeDtypeStruct((V, ROW), jnp.float32),
        scratch_types=[
            pltpu.VMEM((2 * CPP,), jnp.float32),
            pltpu.VMEM((C * C, ROW), jnp.float32),
        ],
    )(cflat)
    run = functools.partial(
        pl.kernel,
        mesh=mesh,
        compiler_params=cp,
        out_type=jax.ShapeDtypeStruct((n,), jnp.float32),
        scratch_types=[
            pltpu.VMEM((2 * CHUNK,), jnp.float32),
            pltpu.VMEM((2 * CHUNK,), jnp.float32),
            pltpu.VMEM((2 * CHUNK,), jnp.float32),
            pltpu.VMEM((2 * CHUNK,), jnp.float32),
            pltpu.VMEM((2 * CHUNK,), jnp.float32),
            pltpu.VMEM((2 * CHUNK,), jnp.float32),
            pltpu.VMEM((2 * CHUNK,), jnp.int32),
            pltpu.VMEM((2 * CHUNK, ROW), jnp.float32),
            pltpu.VMEM((CHUNK,), jnp.float32),
            pltpu.SemaphoreType.DMA,
            pltpu.SemaphoreType.DMA,
        ],
    )(functools.partial(_sc_body, n_chunks))
    out = run(xst[0], xst[1], xst[2], cells)
    return out.reshape(n, 1)


# speedup vs baseline: 8.5538x; 1.1334x over previous
import functools

import jax
import jax.numpy as jnp
from jax import lax
from jax.experimental import pallas as pl
from jax.experimental.pallas import tpu as pltpu
from jax.experimental.pallas import tpu_sc as plsc

D = 256
SCALE = 4
C = D // SCALE
V = C * C * C
ROW = 8
CHUNK = 2048
LANES = 16

NC = 2
NS = 16
NW = NC * NS

CP = 65 * 65
CPP = 4232
CZ_PER_TILE = C // NW

CORNER_OFF = [dz * CPP + dy * 65 + dx
              for dz in (0, 1) for dy in (0, 1) for dx in (0, 1)]


def _sc_build(coarse, cells, cslab_v, slab_v):
    wid = lax.axis_index("s") * NC + lax.axis_index("c")
    iota = lax.iota(jnp.int32, LANES)

    def build_cz(t, _):
        cz = wid * CZ_PER_TILE + t
        pltpu.sync_copy(coarse.at[pl.ds(cz * CPP, 2 * CPP)], cslab_v)

        def build_cy(cy, _):
            for g in range(C // LANES):
                cx = g * LANES + iota
                pos = cy * 65 + cx
                rowi = cy * C + cx
                for j in range(8):
                    vals = plsc.load_gather(cslab_v, [pos + CORNER_OFF[j]])
                    plsc.store_scatter(slab_v, [rowi, jnp.full((LANES,), j,
                                                               jnp.int32)],
                                       vals)
            return 0

        lax.fori_loop(0, C, build_cy, 0)
        pltpu.sync_copy(slab_v, cells.at[pl.ds(cz * (C * C), C * C), :])
        return 0

    lax.fori_loop(0, CZ_PER_TILE, build_cz, 0)


def _sc_body(n_chunks, xs0, xs1, xs2, cells, out,
             x0_v, x1_v, x2_v, f0_v, f1_v, f2_v, idx_v, rows_v, out_v,
             sem_a, sem_b, sem_x0, sem_x1, sem_o0, sem_o1):
    wid = lax.axis_index("s") * NC + lax.axis_index("c")
    iota = lax.iota(jnp.int32, LANES)

    base_w = wid * (n_chunks * CHUNK)
    sems = (sem_a, sem_b)
    sems_x = (sem_x0, sem_x1)
    sems_o = (sem_o0, sem_o1)

    def fire_xs(ch):
        b = (ch % 2) * CHUNK
        base = base_w + ch * CHUNK
        sx = sems_x[ch % 2]
        return [
            pltpu.async_copy(xs0.at[pl.ds(base, CHUNK)],
                             x0_v.at[pl.ds(b, CHUNK)], sx),
            pltpu.async_copy(xs1.at[pl.ds(base, CHUNK)],
                             x1_v.at[pl.ds(b, CHUNK)], sx),
            pltpu.async_copy(xs2.at[pl.ds(base, CHUNK)],
                             x2_v.at[pl.ds(b, CHUNK)], sx),
        ]

    def stage_a(ch, xs_copies):
        b = (ch % 2) * CHUNK
        for cp in xs_copies:
            cp.wait()

        def idx_body(i, _):
            sl = pl.ds(b + i * LANES, LANES)
            s0 = x0_v[sl] * jnp.float32(D)
            s1 = x1_v[sl] * jnp.float32(D)
            s2 = x2_v[sl] * jnp.float32(D)
            i0 = lax.bitwise_and(s0.astype(jnp.int32), -SCALE)
            i1 = lax.bitwise_and(s1.astype(jnp.int32), -SCALE)
            i2 = lax.bitwise_and(s2.astype(jnp.int32), -SCALE)
            f0_v[sl] = (s0 - i0.astype(jnp.float32)) * jnp.float32(1.0 / SCALE)
            f1_v[sl] = (s1 - i1.astype(jnp.float32)) * jnp.float32(1.0 / SCALE)
            f2_v[sl] = (s2 - i2.astype(jnp.float32)) * jnp.float32(1.0 / SCALE)
            c0 = lax.shift_right_arithmetic(i0, 2)
            c1 = lax.shift_right_arithmetic(i1, 2)
            c2 = lax.shift_right_arithmetic(i2, 2)
            idx_v[sl] = (c0 * C + c1) * C + c2
            return 0

        lax.fori_loop(0, CHUNK // LANES, idx_body, 0)

        copies = []
        for j in range(CHUNK // 128):
            isl = pl.ds(b + j * 128, 128)
            copies.append(
                pltpu.async_copy(cells.at[idx_v.at[isl]],
                                 rows_v.at[isl, :], sems[ch % 2]))
        return copies

    def stage_b(ch, copies, prev_out):
        b = (ch % 2) * CHUNK
        base = base_w + ch * CHUNK
        for cp in copies:
            cp.wait()
        if prev_out is not None:
            prev_out.wait()

        def comp_body(i, _):
            sl = pl.ds(b + i * LANES, LANES)
            f0 = f0_v[sl]
            f1 = f1_v[sl]
            f2 = f2_v[sl]
            g0 = jnp.float32(1.0) - f0
            g1 = jnp.float32(1.0) - f1
            g2 = jnp.float32(1.0) - f2
            q00 = g1 * g2
            q10 = f1 * g2
            q01 = g1 * f2
            q11 = f1 * f2
            row = b + i * LANES + iota

            def corner(j):
                col = jnp.full((LANES,), j, jnp.int32)
                return plsc.load_gather(rows_v, [row, col])

            acc = corner(0) * (g0 * q00)
            acc = acc + corner(1) * (f0 * q00)
            acc = acc + corner(2) * (g0 * q10)
            acc = acc + corner(3) * (f0 * q10)
            acc = acc + corner(4) * (g0 * q01)
            acc = acc + corner(5) * (f0 * q01)
            acc = acc + corner(6) * (g0 * q11)
            acc = acc + corner(7) * (f0 * q11)
            out_v[pl.ds(b + i * LANES, LANES)] = acc
            return 0

        lax.fori_loop(0, CHUNK // LANES, comp_body, 0)
        return pltpu.async_copy(out_v.at[pl.ds(b, CHUNK)],
                                out.at[pl.ds(base, CHUNK)], sems_o[ch % 2])

    xsc = {0: fire_xs(0)}
    if n_chunks > 1:
        xsc[1] = fire_xs(1)
    inflight = stage_a(0, xsc[0])
    outc = {}
    for ch in range(n_chunks):
        if ch + 2 < n_chunks:
            xsc[ch + 2] = fire_xs(ch + 2)
        nxt = stage_a(ch + 1, xsc[ch + 1]) if ch + 1 < n_chunks else None
        outc[ch] = stage_b(ch, inflight, outc.get(ch - 2))
        inflight = nxt
    outc[n_chunks - 1].wait()
    outc[n_chunks - 2].wait()


def kernel(xs, data):
    n = xs.shape[0]
    n_chunks = n // (NW * CHUNK)

    a = jnp.concatenate([data[::SCALE], data[D - 1:]], 0)[:, :, :, 0]
    b = jnp.concatenate([a[:, ::SCALE], a[:, D - 1:]], 1)
    c = jnp.concatenate([b[:, :, ::SCALE], b[:, :, D - 1:]], 2)
    cflat = jnp.pad(c.reshape(65, CP), ((0, 1), (0, CPP - CP))).reshape(-1)
    xst = xs.T
    dep = lax.optimization_barrier(cflat[0]) * jnp.float32(0.0)
    xst = xst + dep

    mesh = plsc.VectorSubcoreMesh(core_axis_name="c", subcore_axis_name="s")
    cp = pltpu.CompilerParams(
        needs_layout_passes=False, use_tc_tiling_on_sc=False)
    cells = pl.kernel(
        _sc_build,
        mesh=mesh,
        compiler_params=cp,
        out_type=jax.ShapeDtypeStruct((V, ROW), jnp.float32),
        scratch_types=[
            pltpu.VMEM((2 * CPP,), jnp.float32),
            pltpu.VMEM((C * C, ROW), jnp.float32),
        ],
    )(cflat)
    run = functools.partial(
        pl.kernel,
        mesh=mesh,
        compiler_params=cp,
        out_type=jax.ShapeDtypeStruct((n,), jnp.float32),
        scratch_types=[
            pltpu.VMEM((2 * CHUNK,), jnp.float32),
            pltpu.VMEM((2 * CHUNK,), jnp.float32),
            pltpu.VMEM((2 * CHUNK,), jnp.float32),
            pltpu.VMEM((2 * CHUNK,), jnp.float32),
            pltpu.VMEM((2 * CHUNK,), jnp.float32),
            pltpu.VMEM((2 * CHUNK,), jnp.float32),
            pltpu.VMEM((2 * CHUNK,), jnp.int32),
            pltpu.VMEM((2 * CHUNK, ROW), jnp.float32),
            pltpu.VMEM((2 * CHUNK,), jnp.float32),
            pltpu.SemaphoreType.DMA,
            pltpu.SemaphoreType.DMA,
            pltpu.SemaphoreType.DMA,
            pltpu.SemaphoreType.DMA,
            pltpu.SemaphoreType.DMA,
            pltpu.SemaphoreType.DMA,
        ],
    )(functools.partial(_sc_body, n_chunks))
    out = run(xst[0], xst[1], xst[2], cells)
    return out.reshape(n, 1)


# speedup vs baseline: 12.9074x; 1.5090x over previous
import functools

import jax
import jax.numpy as jnp
from jax import lax
from jax.experimental import pallas as pl
from jax.experimental.pallas import tpu as pltpu
from jax.experimental.pallas import tpu_sc as plsc

D = 256
SCALE = 4
C = D // SCALE
V = C * C * C
ROW = 8
CHUNK = 2048
LANES = 16

NC = 2
NS = 16
NW = NC * NS

CP = 65 * 65
CPP = 4232
CZ_PER_TILE = C // NW

CORNER_OFF = [dz * CPP + dy * 65 + dx
              for dz in (0, 1) for dy in (0, 1) for dx in (0, 1)]


def _sc_build(aflat, cells, stage_v, slab_v, sem):
    wid = lax.axis_index("s") * NC + lax.axis_index("c")
    iota = lax.iota(jnp.int32, LANES)

    for t in range(CZ_PER_TILE):
        cz = wid * CZ_PER_TILE + t
        copies = []
        for zz in range(2):
            zi = cz + zz
            for k in range(65):
                y = min(4 * k, 255)
                copies.append(pltpu.async_copy(
                    aflat.at[pl.ds((zi * 256 + y) * 256, 256)],
                    stage_v.at[pl.ds((zz * 65 + k) * 256, 256)], sem))
        for cp in copies:
            cp.wait()

        def build_cy(cy, _):
            for g in range(C // LANES):
                cx = g * LANES + iota
                x0 = cx * SCALE
                x1 = jnp.minimum(x0 + SCALE, D - 1)
                rowi = cy * C + cx
                for j, (dz, dy, dx) in enumerate(
                        [(dz, dy, dx) for dz in (0, 1) for dy in (0, 1)
                         for dx in (0, 1)]):
                    src = (dz * 65 + cy + dy) * 256 + (x1 if dx else x0)
                    vals = plsc.load_gather(stage_v, [src])
                    plsc.store_scatter(slab_v, [rowi, jnp.full((LANES,), j,
                                                               jnp.int32)],
                                       vals)
            return 0

        lax.fori_loop(0, C, build_cy, 0)
        pltpu.sync_copy(slab_v, cells.at[pl.ds(cz * (C * C), C * C), :])


def _sc_body(n_chunks, xs0, xs1, xs2, cells, out,
             x0_v, x1_v, x2_v, f0_v, f1_v, f2_v, idx_v, rows_v, out_v,
             sem_a, sem_b, sem_x0, sem_x1, sem_o0, sem_o1):
    wid = lax.axis_index("s") * NC + lax.axis_index("c")
    iota = lax.iota(jnp.int32, LANES)

    base_w = wid * (n_chunks * CHUNK)
    sems = (sem_a, sem_b)
    sems_x = (sem_x0, sem_x1)
    sems_o = (sem_o0, sem_o1)

    def fire_xs(ch):
        b = (ch % 2) * CHUNK
        base = base_w + ch * CHUNK
        sx = sems_x[ch % 2]
        return [
            pltpu.async_copy(xs0.at[pl.ds(base, CHUNK)],
                             x0_v.at[pl.ds(b, CHUNK)], sx),
            pltpu.async_copy(xs1.at[pl.ds(base, CHUNK)],
                             x1_v.at[pl.ds(b, CHUNK)], sx),
            pltpu.async_copy(xs2.at[pl.ds(base, CHUNK)],
                             x2_v.at[pl.ds(b, CHUNK)], sx),
        ]

    def stage_a(ch, xs_copies):
        b = (ch % 2) * CHUNK
        for cp in xs_copies:
            cp.wait()

        def idx_body(i, _):
            sl = pl.ds(b + i * LANES, LANES)
            s0 = x0_v[sl] * jnp.float32(D)
            s1 = x1_v[sl] * jnp.float32(D)
            s2 = x2_v[sl] * jnp.float32(D)
            i0 = lax.bitwise_and(s0.astype(jnp.int32), -SCALE)
            i1 = lax.bitwise_and(s1.astype(jnp.int32), -SCALE)
            i2 = lax.bitwise_and(s2.astype(jnp.int32), -SCALE)
            f0_v[sl] = (s0 - i0.astype(jnp.float32)) * jnp.float32(1.0 / SCALE)
            f1_v[sl] = (s1 - i1.astype(jnp.float32)) * jnp.float32(1.0 / SCALE)
            f2_v[sl] = (s2 - i2.astype(jnp.float32)) * jnp.float32(1.0 / SCALE)
            c0 = lax.shift_right_arithmetic(i0, 2)
            c1 = lax.shift_right_arithmetic(i1, 2)
            c2 = lax.shift_right_arithmetic(i2, 2)
            idx_v[sl] = (c0 * C + c1) * C + c2
            return 0

        lax.fori_loop(0, CHUNK // LANES, idx_body, 0)

        copies = []
        for j in range(CHUNK // 128):
            isl = pl.ds(b + j * 128, 128)
            copies.append(
                pltpu.async_copy(cells.at[idx_v.at[isl]],
                                 rows_v.at[isl, :], sems[ch % 2]))
        return copies

    def stage_b(ch, copies, prev_out):
        b = (ch % 2) * CHUNK
        base = base_w + ch * CHUNK
        for cp in copies:
            cp.wait()
        if prev_out is not None:
            prev_out.wait()

        def comp_body(i, _):
            sl = pl.ds(b + i * LANES, LANES)
            f0 = f0_v[sl]
            f1 = f1_v[sl]
            f2 = f2_v[sl]
            g0 = jnp.float32(1.0) - f0
            g1 = jnp.float32(1.0) - f1
            g2 = jnp.float32(1.0) - f2
            q00 = g1 * g2
            q10 = f1 * g2
            q01 = g1 * f2
            q11 = f1 * f2
            row = b + i * LANES + iota

            def corner(j):
                col = jnp.full((LANES,), j, jnp.int32)
                return plsc.load_gather(rows_v, [row, col])

            acc = corner(0) * (g0 * q00)
            acc = acc + corner(1) * (f0 * q00)
            acc = acc + corner(2) * (g0 * q10)
            acc = acc + corner(3) * (f0 * q10)
            acc = acc + corner(4) * (g0 * q01)
            acc = acc + corner(5) * (f0 * q01)
            acc = acc + corner(6) * (g0 * q11)
            acc = acc + corner(7) * (f0 * q11)
            out_v[pl.ds(b + i * LANES, LANES)] = acc
            return 0

        lax.fori_loop(0, CHUNK // LANES, comp_body, 0)
        return pltpu.async_copy(out_v.at[pl.ds(b, CHUNK)],
                                out.at[pl.ds(base, CHUNK)], sems_o[ch % 2])

    xsc = {0: fire_xs(0)}
    if n_chunks > 1:
        xsc[1] = fire_xs(1)
    inflight = stage_a(0, xsc[0])
    outc = {}
    for ch in range(n_chunks):
        if ch + 2 < n_chunks:
            xsc[ch + 2] = fire_xs(ch + 2)
        nxt = stage_a(ch + 1, xsc[ch + 1]) if ch + 1 < n_chunks else None
        outc[ch] = stage_b(ch, inflight, outc.get(ch - 2))
        inflight = nxt
    outc[n_chunks - 1].wait()
    outc[n_chunks - 2].wait()


def kernel(xs, data):
    n = xs.shape[0]
    n_chunks = n // (NW * CHUNK)

    aflat = jnp.concatenate([data[::SCALE], data[D - 1:]],
                            0)[:, :, :, 0].reshape(65 * D * D)
    xst = xs.T
    dep = lax.optimization_barrier(aflat[0]) * jnp.float32(0.0)
    xst = xst + dep

    mesh = plsc.VectorSubcoreMesh(core_axis_name="c", subcore_axis_name="s")
    cp = pltpu.CompilerParams(
        needs_layout_passes=False, use_tc_tiling_on_sc=False)
    cells = pl.kernel(
        _sc_build,
        mesh=mesh,
        compiler_params=cp,
        out_type=jax.ShapeDtypeStruct((V, ROW), jnp.float32),
        scratch_types=[
            pltpu.VMEM((130 * 256,), jnp.float32),
            pltpu.VMEM((C * C, ROW), jnp.float32),
            pltpu.SemaphoreType.DMA,
        ],
    )(aflat)
    run = functools.partial(
        pl.kernel,
        mesh=mesh,
        compiler_params=cp,
        out_type=jax.ShapeDtypeStruct((n,), jnp.float32),
        scratch_types=[
            pltpu.VMEM((2 * CHUNK,), jnp.float32),
            pltpu.VMEM((2 * CHUNK,), jnp.float32),
            pltpu.VMEM((2 * CHUNK,), jnp.float32),
            pltpu.VMEM((2 * CHUNK,), jnp.float32),
            pltpu.VMEM((2 * CHUNK,), jnp.float32),
            pltpu.VMEM((2 * CHUNK,), jnp.float32),
            pltpu.VMEM((2 * CHUNK,), jnp.int32),
            pltpu.VMEM((2 * CHUNK, ROW), jnp.float32),
            pltpu.VMEM((2 * CHUNK,), jnp.float32),
            pltpu.SemaphoreType.DMA,
            pltpu.SemaphoreType.DMA,
            pltpu.SemaphoreType.DMA,
            pltpu.SemaphoreType.DMA,
            pltpu.SemaphoreType.DMA,
            pltpu.SemaphoreType.DMA,
        ],
    )(functools.partial(_sc_body, n_chunks))
    out = run(xst[0], xst[1], xst[2], cells)
    return out.reshape(n, 1)


# speedup vs baseline: 15.7877x; 1.2231x over previous
import functools

import jax
import jax.numpy as jnp
from jax import lax
from jax.experimental import pallas as pl
from jax.experimental.pallas import tpu as pltpu
from jax.experimental.pallas import tpu_sc as plsc

D = 256
SCALE = 4
C = D // SCALE
V = C * C * C
ROW = 8
CHUNK = 2048
LANES = 16

NC = 2
NS = 16
NW = NC * NS

CP = 65 * 65
CPP = 4232
CZ_PER_TILE = C // NW

CORNER_OFF = [dz * CPP + dy * 65 + dx
              for dz in (0, 1) for dy in (0, 1) for dx in (0, 1)]


def _sc_build(dflat, cells, stage_v, slab_v, sem):
    wid = lax.axis_index("s") * NC + lax.axis_index("c")
    iota = lax.iota(jnp.int32, LANES)

    for t in range(CZ_PER_TILE):
        cz = wid * CZ_PER_TILE + t
        copies = []
        for zz in range(2):
            z = jnp.minimum(cz * SCALE + zz * SCALE, D - 1)
            for k in range(65):
                y = min(4 * k, 255)
                copies.append(pltpu.async_copy(
                    dflat.at[pl.ds((z * 256 + y) * 256, 256)],
                    stage_v.at[pl.ds((zz * 65 + k) * 256, 256)], sem))
        for cp in copies:
            cp.wait()

        def build_cy(cy, _):
            for g in range(C // LANES):
                cx = g * LANES + iota
                x0 = cx * SCALE
                x1 = jnp.minimum(x0 + SCALE, D - 1)
                rowi = cy * C + cx
                for j, (dz, dy, dx) in enumerate(
                        [(dz, dy, dx) for dz in (0, 1) for dy in (0, 1)
                         for dx in (0, 1)]):
                    src = (dz * 65 + cy + dy) * 256 + (x1 if dx else x0)
                    vals = plsc.load_gather(stage_v, [src])
                    plsc.store_scatter(slab_v, [rowi, jnp.full((LANES,), j,
                                                               jnp.int32)],
                                       vals)
            return 0

        lax.fori_loop(0, C, build_cy, 0)
        pltpu.sync_copy(slab_v, cells.at[pl.ds(cz * (C * C), C * C), :])


def _sc_body(n_chunks, xs0, xs1, xs2, cells, out,
             x0_v, x1_v, x2_v, f0_v, f1_v, f2_v, idx_v, rows_v, out_v,
             sem_a, sem_b, sem_x0, sem_x1, sem_o0, sem_o1):
    wid = lax.axis_index("s") * NC + lax.axis_index("c")
    iota = lax.iota(jnp.int32, LANES)

    base_w = wid * (n_chunks * CHUNK)
    sems = (sem_a, sem_b)
    sems_x = (sem_x0, sem_x1)
    sems_o = (sem_o0, sem_o1)

    def fire_xs(ch):
        b = (ch % 2) * CHUNK
        base = base_w + ch * CHUNK
        sx = sems_x[ch % 2]
        return [
            pltpu.async_copy(xs0.at[pl.ds(base, CHUNK)],
                             x0_v.at[pl.ds(b, CHUNK)], sx),
            pltpu.async_copy(xs1.at[pl.ds(base, CHUNK)],
                             x1_v.at[pl.ds(b, CHUNK)], sx),
            pltpu.async_copy(xs2.at[pl.ds(base, CHUNK)],
                             x2_v.at[pl.ds(b, CHUNK)], sx),
        ]

    def stage_a(ch, xs_copies):
        b = (ch % 2) * CHUNK
        for cp in xs_copies:
            cp.wait()

        def idx_body(i, _):
            sl = pl.ds(b + i * LANES, LANES)
            s0 = x0_v[sl] * jnp.float32(D)
            s1 = x1_v[sl] * jnp.float32(D)
            s2 = x2_v[sl] * jnp.float32(D)
            i0 = lax.bitwise_and(s0.astype(jnp.int32), -SCALE)
            i1 = lax.bitwise_and(s1.astype(jnp.int32), -SCALE)
            i2 = lax.bitwise_and(s2.astype(jnp.int32), -SCALE)
            f0_v[sl] = (s0 - i0.astype(jnp.float32)) * jnp.float32(1.0 / SCALE)
            f1_v[sl] = (s1 - i1.astype(jnp.float32)) * jnp.float32(1.0 / SCALE)
            f2_v[sl] = (s2 - i2.astype(jnp.float32)) * jnp.float32(1.0 / SCALE)
            c0 = lax.shift_right_arithmetic(i0, 2)
            c1 = lax.shift_right_arithmetic(i1, 2)
            c2 = lax.shift_right_arithmetic(i2, 2)
            idx_v[sl] = (c0 * C + c1) * C + c2
            return 0

        lax.fori_loop(0, CHUNK // LANES, idx_body, 0)

        copies = []
        for j in range(CHUNK // 128):
            isl = pl.ds(b + j * 128, 128)
            copies.append(
                pltpu.async_copy(cells.at[idx_v.at[isl]],
                                 rows_v.at[isl, :], sems[ch % 2]))
        return copies

    def stage_b(ch, copies, prev_out):
        b = (ch % 2) * CHUNK
        base = base_w + ch * CHUNK
        for cp in copies:
            cp.wait()
        if prev_out is not None:
            prev_out.wait()

        def comp_body(i, _):
            sl = pl.ds(b + i * LANES, LANES)
            f0 = f0_v[sl]
            f1 = f1_v[sl]
            f2 = f2_v[sl]
            g0 = jnp.float32(1.0) - f0
            g1 = jnp.float32(1.0) - f1
            g2 = jnp.float32(1.0) - f2
            q00 = g1 * g2
            q10 = f1 * g2
            q01 = g1 * f2
            q11 = f1 * f2
            row = b + i * LANES + iota

            def corner(j):
                col = jnp.full((LANES,), j, jnp.int32)
                return plsc.load_gather(rows_v, [row, col])

            acc = corner(0) * (g0 * q00)
            acc = acc + corner(1) * (f0 * q00)
            acc = acc + corner(2) * (g0 * q10)
            acc = acc + corner(3) * (f0 * q10)
            acc = acc + corner(4) * (g0 * q01)
            acc = acc + corner(5) * (f0 * q01)
            acc = acc + corner(6) * (g0 * q11)
            acc = acc + corner(7) * (f0 * q11)
            out_v[pl.ds(b + i * LANES, LANES)] = acc
            return 0

        lax.fori_loop(0, CHUNK // LANES, comp_body, 0)
        return pltpu.async_copy(out_v.at[pl.ds(b, CHUNK)],
                                out.at[pl.ds(base, CHUNK)], sems_o[ch % 2])

    xsc = {0: fire_xs(0)}
    if n_chunks > 1:
        xsc[1] = fire_xs(1)
    inflight = stage_a(0, xsc[0])
    outc = {}
    for ch in range(n_chunks):
        if ch + 2 < n_chunks:
            xsc[ch + 2] = fire_xs(ch + 2)
        nxt = stage_a(ch + 1, xsc[ch + 1]) if ch + 1 < n_chunks else None
        outc[ch] = stage_b(ch, inflight, outc.get(ch - 2))
        inflight = nxt
    outc[n_chunks - 1].wait()
    outc[n_chunks - 2].wait()


def kernel(xs, data):
    n = xs.shape[0]
    n_chunks = n // (NW * CHUNK)

    dflat = data.reshape(D * D * D)
    xst = xs.T
    dep = lax.optimization_barrier(dflat[0]) * jnp.float32(0.0)
    xst = xst + dep

    mesh = plsc.VectorSubcoreMesh(core_axis_name="c", subcore_axis_name="s")
    cp = pltpu.CompilerParams(
        needs_layout_passes=False, use_tc_tiling_on_sc=False)
    cells = pl.kernel(
        _sc_build,
        mesh=mesh,
        compiler_params=cp,
        out_type=jax.ShapeDtypeStruct((V, ROW), jnp.float32),
        scratch_types=[
            pltpu.VMEM((130 * 256,), jnp.float32),
            pltpu.VMEM((C * C, ROW), jnp.float32),
            pltpu.SemaphoreType.DMA,
        ],
    )(dflat)
    run = functools.partial(
        pl.kernel,
        mesh=mesh,
        compiler_params=cp,
        out_type=jax.ShapeDtypeStruct((n,), jnp.float32),
        scratch_types=[
            pltpu.VMEM((2 * CHUNK,), jnp.float32),
            pltpu.VMEM((2 * CHUNK,), jnp.float32),
            pltpu.VMEM((2 * CHUNK,), jnp.float32),
            pltpu.VMEM((2 * CHUNK,), jnp.float32),
            pltpu.VMEM((2 * CHUNK,), jnp.float32),
            pltpu.VMEM((2 * CHUNK,), jnp.float32),
            pltpu.VMEM((2 * CHUNK,), jnp.int32),
            pltpu.VMEM((2 * CHUNK, ROW), jnp.float32),
            pltpu.VMEM((2 * CHUNK,), jnp.float32),
            pltpu.SemaphoreType.DMA,
            pltpu.SemaphoreType.DMA,
            pltpu.SemaphoreType.DMA,
            pltpu.SemaphoreType.DMA,
            pltpu.SemaphoreType.DMA,
            pltpu.SemaphoreType.DMA,
        ],
    )(functools.partial(_sc_body, n_chunks))
    out = run(xst[0], xst[1], xst[2], cells)
    return out.reshape(n, 1)
